# Initial kernel scaffold; baseline (speedup 1.0000x reference)
import jax
import jax.numpy as jnp
from jax.experimental import pallas as pl


def kernel(token_ids, embedding):
    raise NotImplementedError("write your pallas kernel here")


# speedup vs baseline: 1.8341x; 1.8341x over previous
import functools

import jax
import jax.numpy as jnp
from jax import lax
from jax.experimental import pallas as pl
from jax.experimental.pallas import tpu as pltpu
from jax.experimental.pallas import tpu_sc as plsc

_CHUNK = 512


@functools.lru_cache(maxsize=None)
def _make_gather(B, V, D):
    info = plsc.get_sparse_core_info()
    nc, ns = info.num_cores, info.num_subcores
    nw = nc * ns
    assert B % (nw * _CHUNK) == 0
    b_per_w = B // nw
    n_chunks = b_per_w // _CHUNK
    mesh = plsc.VectorSubcoreMesh(core_axis_name="c", subcore_axis_name="s")

    @functools.partial(
        pl.kernel,
        out_type=jax.ShapeDtypeStruct((B, D), jnp.float32),
        mesh=mesh,
        scratch_types=[
            pltpu.VMEM((b_per_w,), jnp.int32),
            pltpu.VMEM((_CHUNK, D), jnp.float32),
            pltpu.SemaphoreType.DMA,
        ],
        compiler_params=pltpu.CompilerParams(use_tc_tiling_on_sc=False),
    )
    def gather_kernel(table_hbm, idx_hbm, out_hbm, idx_v, rows_v, gsem):
        wid = lax.axis_index("s") * nc + lax.axis_index("c")
        base = wid * b_per_w
        pltpu.sync_copy(idx_hbm.at[pl.ds(base, b_per_w)], idx_v)

        def chunk(g, carry):
            cb = g * _CHUNK
            pltpu.async_copy(
                table_hbm.at[idx_v.at[pl.ds(cb, _CHUNK)]], rows_v, gsem
            ).wait()
            pltpu.sync_copy(rows_v, out_hbm.at[pl.ds(base + cb, _CHUNK)])
            return carry

        lax.fori_loop(0, n_chunks, chunk, 0)

    return gather_kernel


def kernel(token_ids, embedding):
    V, D = embedding.shape
    B = token_ids.shape[0] * token_ids.shape[1]
    idx = token_ids.reshape(-1).astype(jnp.int32)
    out = _make_gather(B, V, D)(embedding, idx)
    return out.reshape(token_ids.shape + (D,))


# speedup vs baseline: 1.8764x; 1.0230x over previous
import functools

import jax
import jax.numpy as jnp
from jax import lax
from jax.experimental import pallas as pl
from jax.experimental.pallas import tpu as pltpu
from jax.experimental.pallas import tpu_sc as plsc

_CHUNK = 640


@functools.lru_cache(maxsize=None)
def _make_gather(B, V, D):
    info = plsc.get_sparse_core_info()
    nc, ns = info.num_cores, info.num_subcores
    nw = nc * ns
    assert B % (nw * _CHUNK) == 0
    b_per_w = B // nw
    n_chunks = b_per_w // _CHUNK
    mesh = plsc.VectorSubcoreMesh(core_axis_name="c", subcore_axis_name="s")

    @functools.partial(
        pl.kernel,
        out_type=jax.ShapeDtypeStruct((B, D), jnp.float32),
        mesh=mesh,
        scratch_types=[
            pltpu.VMEM((b_per_w,), jnp.int32),
            pltpu.VMEM((_CHUNK, D), jnp.float32),
            pltpu.VMEM((_CHUNK, D), jnp.float32),
            pltpu.SemaphoreType.DMA,
            pltpu.SemaphoreType.DMA,
        ],
        compiler_params=pltpu.CompilerParams(use_tc_tiling_on_sc=False),
    )
    def gather_kernel(table_hbm, idx_hbm, out_hbm, idx_v, rows0, rows1, gsem, osem):
        wid = lax.axis_index("s") * nc + lax.axis_index("c")
        base = wid * b_per_w
        pltpu.sync_copy(idx_hbm.at[pl.ds(base, b_per_w)], idx_v)

        def gather_start(g, rows):
            pltpu.async_copy(
                table_hbm.at[idx_v.at[pl.ds(g * _CHUNK, _CHUNK)]], rows, gsem
            )

        def write_start(g, rows):
            pltpu.async_copy(rows, out_hbm.at[pl.ds(base + g * _CHUNK, _CHUNK)], osem)

        def drain_one(sem):
            pltpu.make_async_copy(table_hbm.at[pl.ds(0, _CHUNK)], rows0, sem).wait()

        gather_start(0, rows0)

        def step(g, carry):
            even = g % 2 == 0

            @pl.when(g >= 1)
            def _():
                drain_one(osem)

            @pl.when(jnp.logical_and(g + 1 < n_chunks, even))
            def _():
                gather_start(g + 1, rows1)

            @pl.when(jnp.logical_and(g + 1 < n_chunks, jnp.logical_not(even)))
            def _():
                gather_start(g + 1, rows0)

            drain_one(gsem)

            @pl.when(even)
            def _():
                write_start(g, rows0)

            @pl.when(jnp.logical_not(even))
            def _():
                write_start(g, rows1)

            return carry

        lax.fori_loop(0, n_chunks, step, 0)
        drain_one(osem)

    return gather_kernel


def kernel(token_ids, embedding):
    V, D = embedding.shape
    B = token_ids.shape[0] * token_ids.shape[1]
    idx = token_ids.reshape(-1).astype(jnp.int32)
    out = _make_gather(B, V, D)(embedding, idx)
    return out.reshape(token_ids.shape + (D,))
